# baseline (speedup 1.0000x reference)
import functools

import jax
import jax.numpy as jnp
from jax import lax
from jax.experimental import pallas as pl
from jax.experimental.pallas import tpu as pltpu
from jax.experimental.pallas import tpu_sc as plsc

N_VOCAB = 100000
N_EMBD = 768
N_TOKENS = 2048
BATCH = 4

ROWS = BATCH * N_TOKENS
NC = 2
NS = 16
L = 16
NW = NC * NS
T_PER_W = N_TOKENS // NW
CHUNK = 32
HALVES = T_PER_W // CHUNK
NCHUNK = BATCH * HALVES
VPR = N_EMBD // L
NBUF = 3

_mesh = plsc.VectorSubcoreMesh(core_axis_name="c", subcore_axis_name="s")


@functools.partial(
    pl.kernel,
    mesh=_mesh,
    out_type=jax.ShapeDtypeStruct((ROWS, N_EMBD), jnp.float32),
    scratch_types=[
        pltpu.VMEM((BATCH * T_PER_W,), jnp.int32),
        pltpu.VMEM((CHUNK, N_EMBD), jnp.float32),
        pltpu.VMEM((CHUNK, N_EMBD), jnp.float32),
        pltpu.VMEM((CHUNK, N_EMBD), jnp.float32),
        pltpu.VMEM((T_PER_W, N_EMBD), jnp.float32),
        pltpu.SemaphoreType.DMA,
        pltpu.SemaphoreType.DMA,
        pltpu.SemaphoreType.DMA,
        pltpu.SemaphoreType.DMA,
        pltpu.SemaphoreType.DMA,
        pltpu.SemaphoreType.DMA,
        pltpu.SemaphoreType.DMA,
        pltpu.SemaphoreType.DMA,
    ],
)
def _embed(tokens_hbm, table_hbm, pos_hbm, out_hbm,
           idx_v, rows0, rows1, rows2, pos_v,
           isem, psem, gsem0, gsem1, gsem2, osem0, osem1, osem2):
    wid = lax.axis_index("s") * NC + lax.axis_index("c")
    t0 = wid * T_PER_W
    rows = (rows0, rows1, rows2)
    gsem = (gsem0, gsem1, gsem2)
    osem = (osem0, osem1, osem2)

    pcopy = pltpu.async_copy(pos_hbm.at[pl.ds(t0, T_PER_W)], pos_v, psem)
    icopies = [
        pltpu.async_copy(
            tokens_hbm.at[pl.ds(b * N_TOKENS + t0, T_PER_W)],
            idx_v.at[pl.ds(b * T_PER_W, T_PER_W)], isem)
        for b in range(BATCH)
    ]
    for ic in icopies:
        ic.wait()

    chunks = [(b, h) for b in range(BATCH) for h in range(HALVES)]

    def start_gather(c):
        b, h = chunks[c]
        return pltpu.async_copy(
            table_hbm.at[idx_v.at[pl.ds(b * T_PER_W + h * CHUNK, CHUNK)]],
            rows[c % NBUF], gsem[c % NBUF])

    g_fly = {0: start_gather(0), 1: start_gather(1)}
    o_fly = {}
    pcopy.wait()
    for c in range(NCHUNK):
        b, h = chunks[c]
        buf = c % NBUF
        g_fly.pop(c).wait()
        if c + 2 < NCHUNK:
            if c - 1 in o_fly:
                o_fly.pop(c - 1).wait()
            g_fly[c + 2] = start_gather(c + 2)

        pv = pos_v.at[pl.ds(h * CHUNK, CHUNK)]

        def body(i, _):
            for j in range(VPR):
                sl = pl.ds(j * L, L)
                plsc.addupdate(rows[buf].at[i, sl], pv[i, sl])
            return 0

        lax.fori_loop(0, CHUNK, body, 0)
        o_fly[c] = pltpu.async_copy(
            rows[buf],
            out_hbm.at[pl.ds(b * N_TOKENS + t0 + h * CHUNK, CHUNK)],
            osem[buf])
    for c in list(o_fly):
        o_fly.pop(c).wait()


def kernel(tokens, token_embedding, pos_embedding):
    flat = tokens.reshape(-1).astype(jnp.int32)
    out = _embed(flat, token_embedding, pos_embedding)
    return out.reshape(BATCH, N_TOKENS, N_EMBD)


# speedup vs baseline: 1.0334x; 1.0334x over previous
import functools

import jax
import jax.numpy as jnp
from jax import lax
from jax.experimental import pallas as pl
from jax.experimental.pallas import tpu as pltpu
from jax.experimental.pallas import tpu_sc as plsc

N_VOCAB = 100000
N_EMBD = 768
N_TOKENS = 2048
BATCH = 4

ROWS = BATCH * N_TOKENS
NC = 2
NS = 16
L = 16
NW = NC * NS
T_PER_W = N_TOKENS // NW
CHUNK = 32
HALVES = T_PER_W // CHUNK
NCHUNK = BATCH * HALVES
VPR = N_EMBD // L
NBUF = 3

_mesh = plsc.VectorSubcoreMesh(core_axis_name="c", subcore_axis_name="s")


@functools.partial(
    pl.kernel,
    mesh=_mesh,
    out_type=jax.ShapeDtypeStruct((ROWS, N_EMBD), jnp.float32),
    scratch_types=[
        pltpu.VMEM((BATCH * T_PER_W,), jnp.int32),
        pltpu.VMEM((CHUNK, N_EMBD), jnp.float32),
        pltpu.VMEM((CHUNK, N_EMBD), jnp.float32),
        pltpu.VMEM((CHUNK, N_EMBD), jnp.float32),
        pltpu.VMEM((T_PER_W, N_EMBD), jnp.float32),
        pltpu.SemaphoreType.DMA,
        pltpu.SemaphoreType.DMA,
        pltpu.SemaphoreType.DMA,
        pltpu.SemaphoreType.DMA,
        pltpu.SemaphoreType.DMA,
        pltpu.SemaphoreType.DMA,
        pltpu.SemaphoreType.DMA,
        pltpu.SemaphoreType.DMA,
    ],
)
def _embed(tokens_hbm, table_hbm, pos_hbm, out_hbm,
           idx_v, rows0, rows1, rows2, pos_v,
           isem, psem, gsem0, gsem1, gsem2, osem0, osem1, osem2):
    wid = lax.axis_index("s") * NC + lax.axis_index("c")
    t0 = wid * T_PER_W
    rows = (rows0, rows1, rows2)
    gsem = (gsem0, gsem1, gsem2)
    osem = (osem0, osem1, osem2)

    pcopy = pltpu.async_copy(pos_hbm.at[pl.ds(t0, T_PER_W)], pos_v, psem)
    icopies = [
        pltpu.async_copy(
            tokens_hbm.at[pl.ds(b * N_TOKENS + t0, T_PER_W)],
            idx_v.at[pl.ds(b * T_PER_W, T_PER_W)], isem)
        for b in range(BATCH)
    ]
    for ic in icopies:
        ic.wait()

    chunks = [(b, h) for b in range(BATCH) for h in range(HALVES)]

    def start_gather(c):
        b, h = chunks[c]
        return pltpu.async_copy(
            table_hbm.at[idx_v.at[pl.ds(b * T_PER_W + h * CHUNK, CHUNK)]],
            rows[c % NBUF], gsem[c % NBUF])

    g_fly = {0: start_gather(0), 1: start_gather(1)}
    o_fly = {}
    pcopy.wait()
    for c in range(NCHUNK):
        b, h = chunks[c]
        buf = c % NBUF
        with jax.named_scope(f"g_wait{c}"):
            g_fly.pop(c).wait()
        with jax.named_scope(f"issue{c}"):
            if c + 2 < NCHUNK:
                if c - 1 in o_fly:
                    o_fly.pop(c - 1).wait()
                g_fly[c + 2] = start_gather(c + 2)

        pv = pos_v.at[pl.ds(h * CHUNK, CHUNK)]

        with jax.named_scope(f"add{c}"):
            def body(i, _):
                for j in range(VPR):
                    sl = pl.ds(j * L, L)
                    rows[buf][i, sl] = rows[buf][i, sl] + pv[i, sl]
                return 0

            lax.fori_loop(0, CHUNK, body, 0)
        o_fly[c] = pltpu.async_copy(
            rows[buf],
            out_hbm.at[pl.ds(b * N_TOKENS + t0 + h * CHUNK, CHUNK)],
            osem[buf])
    for c in list(o_fly):
        o_fly.pop(c).wait()


def kernel(tokens, token_embedding, pos_embedding):
    flat = tokens.reshape(-1).astype(jnp.int32)
    out = _embed(flat, token_embedding, pos_embedding)
    return out.reshape(BATCH, N_TOKENS, N_EMBD)


# speedup vs baseline: 1.0348x; 1.0013x over previous
import functools

import jax
import jax.numpy as jnp
from jax import lax
from jax.experimental import pallas as pl
from jax.experimental.pallas import tpu as pltpu
from jax.experimental.pallas import tpu_sc as plsc

N_VOCAB = 100000
N_EMBD = 768
N_TOKENS = 2048
BATCH = 4

ROWS = BATCH * N_TOKENS
NC = 2
NS = 16
NW = NC * NS
T_PER_W = N_TOKENS // NW
CHUNK = 32
HALVES = T_PER_W // CHUNK
NCHUNK = BATCH * HALVES
NBUF = 4

_mesh = plsc.VectorSubcoreMesh(core_axis_name="c", subcore_axis_name="s")


@functools.partial(
    pl.kernel,
    mesh=_mesh,
    out_type=jax.ShapeDtypeStruct((ROWS, N_EMBD), jnp.float32),
    scratch_types=[
        pltpu.VMEM((BATCH * T_PER_W,), jnp.int32),
        pltpu.VMEM((CHUNK, N_EMBD), jnp.float32),
        pltpu.VMEM((CHUNK, N_EMBD), jnp.float32),
        pltpu.VMEM((CHUNK, N_EMBD), jnp.float32),
        pltpu.VMEM((CHUNK, N_EMBD), jnp.float32),
        pltpu.SemaphoreType.DMA,
        pltpu.SemaphoreType.DMA,
        pltpu.SemaphoreType.DMA,
        pltpu.SemaphoreType.DMA,
        pltpu.SemaphoreType.DMA,
        pltpu.SemaphoreType.DMA,
        pltpu.SemaphoreType.DMA,
        pltpu.SemaphoreType.DMA,
        pltpu.SemaphoreType.DMA,
        pltpu.SemaphoreType.DMA,
        pltpu.SemaphoreType.DMA,
        pltpu.SemaphoreType.DMA,
        pltpu.SemaphoreType.DMA,
    ],
)
def _embed(tokens_hbm, table_hbm, pos_hbm, out_hbm,
           idx_v, rows0, rows1, rows2, rows3,
           isem,
           fsem0, fsem1, fsem2, fsem3,
           gsem0, gsem1, gsem2, gsem3,
           osem0, osem1, osem2, osem3):
    wid = lax.axis_index("s") * NC + lax.axis_index("c")
    t0 = wid * T_PER_W
    rows = (rows0, rows1, rows2, rows3)
    fsem = (fsem0, fsem1, fsem2, fsem3)
    gsem = (gsem0, gsem1, gsem2, gsem3)
    osem = (osem0, osem1, osem2, osem3)

    icopies = [
        pltpu.async_copy(
            tokens_hbm.at[pl.ds(b * N_TOKENS + t0, T_PER_W)],
            idx_v.at[pl.ds(b * T_PER_W, T_PER_W)], isem)
        for b in range(BATCH)
    ]
    for ic in icopies:
        ic.wait()

    chunks = [(b, h) for b in range(BATCH) for h in range(HALVES)]

    def start_fill(c):
        _, h = chunks[c]
        return pltpu.async_copy(
            pos_hbm.at[pl.ds(t0 + h * CHUNK, CHUNK)], rows[c % NBUF],
            fsem[c % NBUF])

    def start_gather(c):
        b, h = chunks[c]
        return pltpu.async_copy(
            table_hbm.at[idx_v.at[pl.ds(b * T_PER_W + h * CHUNK, CHUNK)]],
            rows[c % NBUF], gsem[c % NBUF], add=True)

    def start_drain(c):
        b, h = chunks[c]
        return pltpu.async_copy(
            rows[c % NBUF],
            out_hbm.at[pl.ds(b * N_TOKENS + t0 + h * CHUNK, CHUNK)],
            osem[c % NBUF])

    f_fly = {c: start_fill(c) for c in range(NBUF)}
    g_fly = {}
    o_fly = {}
    for c in range(2):
        f_fly.pop(c).wait()
        g_fly[c] = start_gather(c)

    for c in range(NCHUNK):
        with jax.named_scope(f"g_wait{c}"):
            g_fly.pop(c).wait()
        o_fly[c] = start_drain(c)
        with jax.named_scope(f"issue{c}"):
            if c + 2 < NCHUNK:
                f_fly.pop(c + 2).wait()
                g_fly[c + 2] = start_gather(c + 2)
            if c + NBUF < NCHUNK:
                o_fly.pop(c).wait()
                f_fly[c + NBUF] = start_fill(c + NBUF)
    for c in list(o_fly):
        o_fly.pop(c).wait()


def kernel(tokens, token_embedding, pos_embedding):
    flat = tokens.reshape(-1).astype(jnp.int32)
    out = _embed(flat, token_embedding, pos_embedding)
    return out.reshape(BATCH, N_TOKENS, N_EMBD)
